# Initial kernel scaffold; baseline (speedup 1.0000x reference)
import jax
import jax.numpy as jnp
from jax.experimental import pallas as pl


def kernel(x, emb, W1, b1, W2, b2):
    raise NotImplementedError("write your pallas kernel here")


# speedup vs baseline: 7.5425x; 7.5425x over previous
import functools

import jax
import jax.numpy as jnp
from jax import lax
from jax.experimental import pallas as pl
from jax.experimental.pallas import tpu as pltpu
from jax.experimental.pallas import tpu_sc as plsc

BATCH = 16384
HIST = 200
EMB_DIM = 16

_info = plsc.get_sparse_core_info()
_NC, _NS = _info.num_cores, _info.num_subcores
_NW = _NC * _NS
_ROWS_PER_W = BATCH // _NW
_CHUNK_B = 16
_CHUNK_IDX = _CHUNK_B * HIST
_IDX_ROWS = _CHUNK_IDX // 128
_N_CHUNKS = _ROWS_PER_W // _CHUNK_B


def _sc_pool_body(x1d_hbm, emb_hbm, out_hbm, idx_v, rows_v, acc_v, sem):
    wid = lax.axis_index("s") * _NC + lax.axis_index("c")
    flat_base = wid * _ROWS_PER_W * HIST

    def chunk_body(c, _):
        pltpu.sync_copy(x1d_hbm.at[pl.ds(flat_base + c * _CHUNK_IDX, _CHUNK_IDX)],
                        idx_v)

        def fire(j, _):
            pltpu.async_copy(emb_hbm.at[idx_v.at[pl.ds(j * 128, 128)]],
                             rows_v.at[pl.ds(j * 128, 128), :], sem)
            return 0
        lax.fori_loop(0, _IDX_ROWS, fire, 0)
        pltpu.make_async_copy(emb_hbm.at[pl.ds(0, _CHUNK_IDX), :], rows_v,
                              sem).wait()

        def row_body(b, _):
            def add8(j, acc):
                base = b * HIST + j * 8
                for k in range(8):
                    acc = acc + rows_v[base + k, :]
                return acc
            acc = lax.fori_loop(0, HIST // 8, add8,
                                jnp.zeros((EMB_DIM,), jnp.float32))
            acc_v[c * _CHUNK_B + b, :] = acc
            return 0
        lax.fori_loop(0, _CHUNK_B, row_body, 0)
        return 0

    lax.fori_loop(0, _N_CHUNKS, chunk_body, 0)
    pltpu.sync_copy(acc_v, out_hbm.at[pl.ds(wid * _ROWS_PER_W, _ROWS_PER_W), :])


_sc_pool = pl.kernel(
    _sc_pool_body,
    out_type=jax.ShapeDtypeStruct((BATCH, EMB_DIM), jnp.float32),
    mesh=plsc.VectorSubcoreMesh(core_axis_name="c", subcore_axis_name="s"),
    scratch_types=[
        pltpu.VMEM((_CHUNK_IDX,), jnp.int32),
        pltpu.VMEM((_CHUNK_IDX, EMB_DIM), jnp.float32),
        pltpu.VMEM((_ROWS_PER_W, EMB_DIM), jnp.float32),
        pltpu.SemaphoreType.DMA,
    ],
    compiler_params=pltpu.CompilerParams(use_tc_tiling_on_sc=False),
)


def _mlp_body(p_ref, w1_ref, b1_ref, w2_ref, b2_ref, o_ref):
    h = jnp.dot(p_ref[...], w1_ref[...], preferred_element_type=jnp.float32)
    h = jnp.maximum(h + b1_ref[...], 0.0)
    o_ref[...] = (jnp.dot(h, w2_ref[...], preferred_element_type=jnp.float32)
                  + b2_ref[...])


_MLP_BLK = 2048


def _mlp(pooled, w1t, b1, w2t, b2):
    grid = (BATCH // _MLP_BLK,)
    return pl.pallas_call(
        _mlp_body,
        grid=grid,
        in_specs=[
            pl.BlockSpec((_MLP_BLK, EMB_DIM), lambda i: (i, 0)),
            pl.BlockSpec((EMB_DIM, EMB_DIM), lambda i: (0, 0)),
            pl.BlockSpec((1, EMB_DIM), lambda i: (0, 0)),
            pl.BlockSpec((EMB_DIM, 2), lambda i: (0, 0)),
            pl.BlockSpec((1, 2), lambda i: (0, 0)),
        ],
        out_specs=pl.BlockSpec((_MLP_BLK, 2), lambda i: (i, 0)),
        out_shape=jax.ShapeDtypeStruct((BATCH, 2), jnp.float32),
    )(pooled, w1t, b1, w2t, b2)


def kernel(x, emb, W1, b1, W2, b2):
    x1d = x.astype(jnp.int32).reshape(-1)
    pooled = _sc_pool(x1d, emb)
    w1t = (W1 * (1.0 / HIST)).T
    return _mlp(pooled, w1t, b1[None, :], W2.T, b2[None, :])


# speedup vs baseline: 9.0343x; 1.1978x over previous
import functools

import jax
import jax.numpy as jnp
from jax import lax
from jax.experimental import pallas as pl
from jax.experimental.pallas import tpu as pltpu
from jax.experimental.pallas import tpu_sc as plsc

BATCH = 16384
HIST = 200
EMB_DIM = 16

_info = plsc.get_sparse_core_info()
_NC, _NS = _info.num_cores, _info.num_subcores
_NW = _NC * _NS
_ROWS_PER_W = BATCH // _NW
_CHUNK_B = 16
_CHUNK_IDX = _CHUNK_B * HIST
_IDX_ROWS = _CHUNK_IDX // 128
_N_CHUNKS = _ROWS_PER_W // _CHUNK_B


def _sc_pool_body(x1d_hbm, emb_hbm, out_hbm,
                  idx_v0, idx_v1, rows_v0, rows_v1, acc_v, sem0, sem1):
    wid = lax.axis_index("s") * _NC + lax.axis_index("c")
    flat_base = wid * _ROWS_PER_W * HIST

    def stage_and_fire(c, idx_v, rows_v, sem):
        pltpu.sync_copy(x1d_hbm.at[pl.ds(flat_base + c * _CHUNK_IDX, _CHUNK_IDX)],
                        idx_v)

        def fire(j, _):
            pltpu.async_copy(emb_hbm.at[idx_v.at[pl.ds(j * 128, 128)]],
                             rows_v.at[pl.ds(j * 128, 128), :], sem)
            return 0
        lax.fori_loop(0, _IDX_ROWS, fire, 0)

    def drain(rows_v, sem):
        pltpu.make_async_copy(emb_hbm.at[pl.ds(0, _CHUNK_IDX), :], rows_v,
                              sem).wait()

    def accum(c, rows_v):
        def row_body(b, _):
            def add8(j, acc):
                base = b * HIST + j * 8
                for k in range(8):
                    acc = acc + rows_v[base + k, :]
                return acc
            acc = lax.fori_loop(0, HIST // 8, add8,
                                jnp.zeros((EMB_DIM,), jnp.float32))
            acc_v[c * _CHUNK_B + b, :] = acc
            return 0
        lax.fori_loop(0, _CHUNK_B, row_body, 0)

    stage_and_fire(0, idx_v0, rows_v0, sem0)

    def pair_body(p, _):
        c0 = p * 2
        stage_and_fire(c0 + 1, idx_v1, rows_v1, sem1)
        drain(rows_v0, sem0)
        accum(c0, rows_v0)

        @pl.when(p < _N_CHUNKS // 2 - 1)
        def _():
            stage_and_fire(c0 + 2, idx_v0, rows_v0, sem0)
        drain(rows_v1, sem1)
        accum(c0 + 1, rows_v1)
        return 0

    lax.fori_loop(0, _N_CHUNKS // 2, pair_body, 0)
    pltpu.sync_copy(acc_v, out_hbm.at[pl.ds(wid * _ROWS_PER_W, _ROWS_PER_W), :])


_sc_pool = pl.kernel(
    _sc_pool_body,
    out_type=jax.ShapeDtypeStruct((BATCH, EMB_DIM), jnp.float32),
    mesh=plsc.VectorSubcoreMesh(core_axis_name="c", subcore_axis_name="s"),
    scratch_types=[
        pltpu.VMEM((_CHUNK_IDX,), jnp.int32),
        pltpu.VMEM((_CHUNK_IDX,), jnp.int32),
        pltpu.VMEM((_CHUNK_IDX, EMB_DIM), jnp.float32),
        pltpu.VMEM((_CHUNK_IDX, EMB_DIM), jnp.float32),
        pltpu.VMEM((_ROWS_PER_W, EMB_DIM), jnp.float32),
        pltpu.SemaphoreType.DMA,
        pltpu.SemaphoreType.DMA,
    ],
    compiler_params=pltpu.CompilerParams(use_tc_tiling_on_sc=False),
)


def _mlp_body(p_ref, w1_ref, b1_ref, w2_ref, b2_ref, o_ref):
    h = jnp.dot(p_ref[...], w1_ref[...], preferred_element_type=jnp.float32)
    h = jnp.maximum(h + b1_ref[...], 0.0)
    o_ref[...] = (jnp.dot(h, w2_ref[...], preferred_element_type=jnp.float32)
                  + b2_ref[...])


_MLP_BLK = 2048


def _mlp(pooled, w1t, b1, w2t, b2):
    grid = (BATCH // _MLP_BLK,)
    return pl.pallas_call(
        _mlp_body,
        grid=grid,
        in_specs=[
            pl.BlockSpec((_MLP_BLK, EMB_DIM), lambda i: (i, 0)),
            pl.BlockSpec((EMB_DIM, EMB_DIM), lambda i: (0, 0)),
            pl.BlockSpec((1, EMB_DIM), lambda i: (0, 0)),
            pl.BlockSpec((EMB_DIM, 2), lambda i: (0, 0)),
            pl.BlockSpec((1, 2), lambda i: (0, 0)),
        ],
        out_specs=pl.BlockSpec((_MLP_BLK, 2), lambda i: (i, 0)),
        out_shape=jax.ShapeDtypeStruct((BATCH, 2), jnp.float32),
    )(pooled, w1t, b1, w2t, b2)


def kernel(x, emb, W1, b1, W2, b2):
    x1d = x.astype(jnp.int32).reshape(-1)
    pooled = _sc_pool(x1d, emb)
    w1t = (W1 * (1.0 / HIST)).T
    return _mlp(pooled, w1t, b1[None, :], W2.T, b2[None, :])
